# baseline (speedup 1.0000x reference)
import functools
import jax
import jax.numpy as jnp
from jax import lax
from jax.experimental import pallas as pl
from jax.experimental.pallas import tpu as pltpu, tpu_sc as plsc

C = 256
H = 100
W = 100
Z = 8
WZ = W * Z
PER_C = H * WZ
NW = 32
CPW = C // NW
IPC = 5
COLS = IPC * WZ
N_CHUNK = PER_C // COLS
WPC = 224

_mesh = plsc.VectorSubcoreMesh(core_axis_name="c", subcore_axis_name="s")


@functools.partial(
    pl.kernel,
    mesh=_mesh,
    compiler_params=pltpu.CompilerParams(needs_layout_passes=False),
    out_type=jax.ShapeDtypeStruct((C * PER_C,), jnp.float32),
    scratch_types=[
        pltpu.VMEM((CPW * WPC,), jnp.float32),
        pltpu.VMEM((CPW * WZ,), jnp.float32),
        pltpu.VMEM((2 * CPW * COLS,), jnp.float32),
        pltpu.SemaphoreType.DMA,
        pltpu.SemaphoreType.DMA,
    ],
)
def _sc_kernel(wpack_hbm, out_hbm, wv, rz, buf, sem_w, sem0):
    wid = lax.axis_index("s") * 2 + lax.axis_index("c")
    c0 = wid * CPW
    iota = lax.iota(jnp.int32, 16)
    iota_d8 = iota >> 3
    zeros16 = jnp.zeros((16,), jnp.int32)

    pltpu.async_copy(wpack_hbm.at[pl.ds(c0 * WPC, CPW * WPC)], wv, sem_w).wait()

    def rz_body(ch, carry):
        wb = ch * WPC
        z16 = wv[pl.ds(wb + 200, 16)]
        for v in range(WZ // 16):
            rv = plsc.load_gather(wv, [iota_d8 + (wb + 100 + 2 * v)])
            rz[pl.ds(ch * WZ + 16 * v, 16)] = rv + z16
        return carry

    lax.fori_loop(0, CPW, rz_body, 0)

    def chunk_copies(u, b, start):
        handles = []
        for ch in range(CPW):
            src = buf.at[pl.ds(b * (CPW * COLS) + ch * COLS, COLS)]
            dst = out_hbm.at[pl.ds((c0 + ch) * PER_C + u * COLS, COLS)]
            if start:
                handles.append(pltpu.async_copy(src, dst, sem0))
            else:
                handles.append(pltpu.make_async_copy(src, dst, sem0))
        return handles

    def chunk_body(u, carry):
        b = lax.rem(u, 2)

        @pl.when(u >= 2)
        def _():
            for h in chunk_copies(u, b, start=False):
                h.wait()

        def fill(ch, carry2):
            bb = b * (CPW * COLS) + ch * COLS
            for ii in range(IPC):
                colv = plsc.load_gather(
                    wv, [zeros16 + (ch * WPC + u * IPC + ii)]
                )
                for v in range(WZ // 16):
                    buf[pl.ds(bb + ii * WZ + 16 * v, 16)] = (
                        rz[pl.ds(ch * WZ + 16 * v, 16)] + colv
                    )
            return carry2

        lax.fori_loop(0, CPW, fill, 0)
        chunk_copies(u, b, start=True)
        return carry

    lax.fori_loop(0, N_CHUNK, chunk_body, 0)
    for h in chunk_copies(0, 0, start=False):
        h.wait()
    for h in chunk_copies(1, 1, start=False):
        h.wait()


def kernel(row_weight, col_weight, z_weight, bs, h, w, z):
    col_t = col_weight.T
    row_t = row_weight.T
    z_t = z_weight.T
    wpack = jnp.concatenate(
        [col_t, row_t, z_t, z_t, jnp.zeros((C, 8), jnp.float32)], axis=1
    )
    out1 = _sc_kernel(wpack.reshape(-1))
    return out1.reshape(1, C, H, W, Z)


# speedup vs baseline: 56.6036x; 56.6036x over previous
import jax
import jax.numpy as jnp
from jax.experimental import pallas as pl

C = 256
H = 100
W = 100
Z = 8
H_BLK = 8


def _body(col_ref, row_ref, z_ref, out_ref):
    col_b = col_ref[...]
    row_b = row_ref[...]
    z_b = z_ref[...]
    out_ref[...] = (
        col_b[:, None, None, :] + row_b[None, :, None, :] + z_b[None, None, :, :]
    )


def kernel(row_weight, col_weight, z_weight, bs, h, w, z):
    pos4 = pl.pallas_call(
        _body,
        grid=(pl.cdiv(H, H_BLK),),
        in_specs=[
            pl.BlockSpec((H_BLK, C), lambda i: (i, 0)),
            pl.BlockSpec((W, C), lambda i: (0, 0)),
            pl.BlockSpec((Z, C), lambda i: (0, 0)),
        ],
        out_specs=pl.BlockSpec((H_BLK, W, Z, C), lambda i: (i, 0, 0, 0)),
        out_shape=jax.ShapeDtypeStruct((H, W, Z, C), jnp.float32),
    )(col_weight, row_weight, z_weight)
    return jnp.transpose(pos4, (3, 0, 1, 2))[None]
